# baseline (speedup 1.0000x reference)
import functools

import jax
import jax.numpy as jnp
import numpy as np
from jax import lax
from jax.experimental import pallas as pl
from jax.experimental.pallas import tpu as pltpu
from jax.experimental.pallas import tpu_sc as plsc

N = 10000
E = 320000
D = 128
G = 64
FC = 256
OUT = 10

NC = 2
NS = 16
NW = NC * NS
EPT = E // NW
CHUNK = 80
NCHUNK = EPT // CHUNK
RPT = 624
RPT0 = 16
NLANE = D // 16

_PERM = np.zeros((D,), np.int32)
for _b in range(D // 32):
  for _h in range(2):
    for _t in range(16):
      _PERM[_b * 32 + _h * 16 + _t] = _b * 32 + 2 * _t + _h


def _make_spmm(interpret=False):
  mesh = plsc.VectorSubcoreMesh(core_axis_name="c", subcore_axis_name="s")

  @functools.partial(
      pl.kernel,
      out_type=jax.ShapeDtypeStruct((NC, N, D), jnp.float32),
      mesh=mesh,
      scratch_types=[
          pltpu.VMEM((EPT,), jnp.int32),
          pltpu.VMEM((CHUNK,), jnp.float32),
          pltpu.VMEM((CHUNK,), jnp.float32),
          pltpu.VMEM((CHUNK,), jnp.int32),
          pltpu.VMEM((CHUNK,), jnp.int32),
          pltpu.VMEM((CHUNK,), jnp.int32),
          pltpu.VMEM((CHUNK,), jnp.int32),
          pltpu.VMEM((CHUNK,), jnp.int32),
          pltpu.VMEM((CHUNK,), jnp.int32),
          pltpu.VMEM((CHUNK, D // 2), jnp.int32),
          pltpu.VMEM((CHUNK, D // 2), jnp.int32),
          pltpu.VMEM((CHUNK, D), jnp.float32),
          pltpu.VMEM((CHUNK, D), jnp.float32),
          pltpu.VMEM_SHARED((N, D), jnp.float32),
          pltpu.SemaphoreType.DMA,
          pltpu.SemaphoreType.DMA,
          pltpu.SemaphoreType.DMA,
          pltpu.SemaphoreType.DMA,
          pltpu.SemaphoreType.DMA,
      ],
      compiler_params=pltpu.CompilerParams(needs_layout_passes=False,
                                           use_tc_tiling_on_sc=False),
      interpret=interpret,
  )
  def spmm(x_hbm, pk_hbm, w_hbm, out_hbm,
           pk_v, w0, w1, src0, src1, dst0, dst1, dst2, dst3,
           bf0, bf1, f0, f1, acc,
           semi, semg0, semg1, sems0, sems1):
    c = lax.axis_index("c")
    s = lax.axis_index("s")
    wid = c * NS + s
    ebase = wid * EPT

    dpk = pltpu.async_copy(pk_hbm.at[pl.ds(ebase, EPT)], pk_v, semi)

    zero16 = jnp.zeros((16,), jnp.float32)
    wv_ = (w0, w1)
    srcc = (src0, src1)
    dstc = (dst0, dst1, dst2, dst3)
    bfb = (bf0, bf1)
    fb = (f0, f1)
    semg = (semg0, semg1)
    sems = (sems0, sems1)

    @pl.loop(0, CHUNK)
    def _zero_fill(r):
      for j in range(NLANE):
        f0[r, pl.ds(j * 16, 16)] = zero16
        f1[r, pl.ds(j * 16, 16)] = zero16

    row0 = RPT0 + s * RPT
    zcopies = []
    for k in range(RPT // CHUNK):
      zcopies.append(pltpu.async_copy(
          f0, acc.at[pl.ds(row0 + k * CHUNK, CHUNK)], semg0))
    rem = RPT - (RPT // CHUNK) * CHUNK
    zcopies.append(pltpu.async_copy(
        f0.at[pl.ds(0, rem)], acc.at[pl.ds(row0 + RPT - rem, rem)],
        semg0))

    @pl.when(s == 0)
    def _zero_head():
      pltpu.sync_copy(f0.at[pl.ds(0, RPT0)], acc.at[pl.ds(0, RPT0)])

    for d in zcopies:
      d.wait()
    dpk.wait()
    plsc.subcore_barrier()

    def unpack(k, si, di):
      for g in range(CHUNK // 16):
        v = pk_v[pl.ds(k * CHUNK + g * 16, 16)]
        sl = pl.ds(g * 16, 16)
        srcc[si][sl] = jnp.bitwise_and(v, 0xFFFF)
        dstc[di][sl] = lax.shift_right_logical(v, 16)

    def issue_gather(k, r):
      pltpu.async_copy(x_hbm.at[srcc[r]], bfb[r], semg[r])
      pltpu.async_copy(w_hbm.at[pl.ds(ebase + k * CHUNK, CHUNK)],
                       wv_[r], semg[r])

    def wait_gather(r):
      pltpu.make_async_copy(x_hbm.at[pl.ds(0, CHUNK)], bfb[r],
                            semg[r]).wait()
      pltpu.make_async_copy(w_hbm.at[pl.ds(0, CHUNK)], wv_[r],
                            semg[r]).wait()

    def issue_scatter(r, d):
      pltpu.async_copy(fb[r], acc.at[dstc[d]], sems[r], add=True)

    def wait_scatter(r, d):
      pltpu.make_async_copy(fb[r], acc.at[dstc[d]], sems[r]).wait()

    def scale(r):
      rbf = bfb[r]
      wref = wv_[r]
      rf = fb[r]
      hi_mask = jnp.full((16,), -65536, jnp.int32)

      @pl.loop(0, CHUNK // 16)
      def _scale(g):
        wgrp = wref[pl.ds(g * 16, 16)]
        for e in range(16):
          wv = wgrp.at[jnp.full((16,), e, jnp.int32)].get(
              mode="promise_in_bounds")
          i = g * 16 + e
          for b in range(D // 32):
            v = rbf[i, pl.ds(b * 16, 16)]
            lo = plsc.bitcast(lax.shift_left(v, 16), jnp.float32)
            hi = plsc.bitcast(jnp.bitwise_and(v, hi_mask), jnp.float32)
            rf[i, pl.ds(b * 32, 16)] = lo * wv
            rf[i, pl.ds(b * 32 + 16, 16)] = hi * wv

    def step(k, r, rn, d, dp, dn):
      wait_scatter(r, dp)
      unpack(k + 1, rn, dn)
      issue_gather(k + 1, rn)
      wait_gather(r)
      scale(r)
      issue_scatter(r, d)

    unpack(0, 1, 2)
    issue_scatter(0, 2)
    unpack(0, 1, 3)
    issue_scatter(1, 3)
    unpack(0, 0, 0)
    issue_gather(0, 0)

    @pl.loop(0, (NCHUNK - 5) // 4)
    def _quad(i):
      k0 = i * 4
      step(k0, 0, 1, 0, 2, 1)
      step(k0 + 1, 1, 0, 1, 3, 2)
      step(k0 + 2, 0, 1, 2, 0, 3)
      step(k0 + 3, 1, 0, 3, 1, 0)

    step(NCHUNK - 5, 0, 1, 0, 2, 1)
    step(NCHUNK - 4, 1, 0, 1, 3, 2)
    step(NCHUNK - 3, 0, 1, 2, 0, 3)
    step(NCHUNK - 2, 1, 0, 3, 1, 0)
    wait_scatter(0, 2)
    wait_gather(0)
    scale(0)
    issue_scatter(0, 0)
    wait_scatter(1, 3)
    wait_scatter(0, 0)

    plsc.subcore_barrier()
    pltpu.sync_copy(acc.at[pl.ds(row0, RPT)],
                    out_hbm.at[c, pl.ds(row0, RPT)])

    @pl.when(s == 0)
    def _write_head():
      pltpu.sync_copy(acc.at[pl.ds(0, RPT0)], out_hbm.at[c, pl.ds(0, RPT0)])

  return spmm


def _dense1_body(aggp, x, wrel, brel, wroot, g, b, out, outb):
  agg = aggp[0] + aggp[1]
  z = lax.dot_general(agg, wrel[...], (((1,), (1,)), ((), ())),
                      precision=lax.Precision.HIGHEST)
  z = z + brel[...]
  z = z + lax.dot_general(x[...], wroot[...], (((1,), (1,)), ((), ())),
                          precision=lax.Precision.HIGHEST)
  z = jnp.maximum(z, 0.0)
  mu = jnp.mean(z, axis=0, keepdims=True)
  var = jnp.mean((z - mu) ** 2, axis=0, keepdims=True)
  h = g[...] * (z - mu) / jnp.sqrt(var + 1e-5) + b[...]
  out[...] = h
  outb[...] = h.astype(jnp.bfloat16)


def _dense2_body(aggp, h1, wrel, brel, wroot, g2, b2, mem2d,
                 fc1w, fc1b, g3, b3, fc2w, fc2b, out):
  agg = aggp[0] + aggp[1]
  z = lax.dot_general(agg, wrel[...], (((1,), (1,)), ((), ())),
                      precision=lax.Precision.HIGHEST)
  z = z + brel[...]
  z = z + lax.dot_general(h1[...], wroot[...], (((1,), (1,)), ((), ())),
                          precision=lax.Precision.HIGHEST)
  z = jnp.maximum(z, 0.0)
  mu = jnp.mean(z, axis=0, keepdims=True)
  var = jnp.mean((z - mu) ** 2, axis=0, keepdims=True)
  h2 = g2[...] * (z - mu) / jnp.sqrt(var + 1e-5) + b2[...]

  gids = lax.broadcasted_iota(jnp.int32, (N, G), 1)
  oh = (mem2d[...] == gids).astype(jnp.float32)
  ssum = lax.dot_general(oh, h2, (((0,), (0,)), ((), ())),
                         precision=lax.Precision.HIGHEST)
  cnt = lax.dot_general(oh, jnp.ones((N, 1), jnp.float32),
                        (((0,), (0,)), ((), ())),
                        precision=lax.Precision.HIGHEST)
  pooled = ssum / jnp.maximum(cnt, 1.0)

  a = lax.dot_general(pooled, fc1w[...], (((1,), (1,)), ((), ())),
                      precision=lax.Precision.HIGHEST)
  a = jnp.maximum(a + fc1b[...], 0.0)
  mu3 = jnp.mean(a, axis=0, keepdims=True)
  var3 = jnp.mean((a - mu3) ** 2, axis=0, keepdims=True)
  a = g3[...] * (a - mu3) / jnp.sqrt(var3 + 1e-5) + b3[...]

  logits = lax.dot_general(a, fc2w[...], (((1,), (1,)), ((), ())),
                           precision=lax.Precision.HIGHEST)
  out[...] = logits + fc2b[...]


_TC_PARAMS = pltpu.CompilerParams(vmem_limit_bytes=100 * 1024 * 1024)


def _make_dense1(interpret=False):
  return pl.pallas_call(
      _dense1_body,
      out_shape=(jax.ShapeDtypeStruct((N, D), jnp.float32),
                 jax.ShapeDtypeStruct((N, D), jnp.bfloat16)),
      compiler_params=_TC_PARAMS,
      interpret=interpret,
  )


def _make_dense2(interpret=False):
  return pl.pallas_call(
      _dense2_body,
      out_shape=jax.ShapeDtypeStruct((G, OUT), jnp.float32),
      compiler_params=_TC_PARAMS,
      interpret=interpret,
  )


def kernel(x, membership, edges, weights, W_rel1, b_rel1, W_root1,
           W_rel2, b_rel2, W_root2, bn1_g, bn1_b, bn2_g, bn2_b,
           fc1_W, fc1_b, bn3_g, bn3_b, fc2_W, fc2_b):
  packed = jnp.bitwise_or(edges[0], jnp.left_shift(edges[1], 16))
  mem2d = membership.reshape(N, 1)

  spmm = _make_spmm()
  dense1 = _make_dense1()
  dense2 = _make_dense2()

  xb = x.astype(jnp.bfloat16)
  x1i = lax.bitcast_convert_type(xb.reshape(N, D // 2, 2), jnp.int32)
  aggp1 = spmm(x1i, packed, weights)
  h1, h1b = dense1(aggp1, x, W_rel1[:, _PERM], b_rel1.reshape(1, D),
                   W_root1, bn1_g.reshape(1, D), bn1_b.reshape(1, D))
  h1i = lax.bitcast_convert_type(h1b.reshape(N, D // 2, 2), jnp.int32)
  aggp2 = spmm(h1i, packed, weights)
  logits = dense2(aggp2, h1, W_rel2[:, _PERM], b_rel2.reshape(1, D),
                  W_root2,
                  bn2_g.reshape(1, D), bn2_b.reshape(1, D), mem2d,
                  fc1_W, fc1_b.reshape(1, FC), bn3_g.reshape(1, FC),
                  bn3_b.reshape(1, FC), fc2_W, fc2_b.reshape(1, OUT))
  return logits


# speedup vs baseline: 2.0596x; 2.0596x over previous
import functools

import jax
import jax.numpy as jnp
from jax import lax
from jax.experimental import pallas as pl
from jax.experimental.pallas import tpu as pltpu
from jax.experimental.pallas import tpu_sc as plsc

N = 10000
E = 320000
D = 128
G = 64
FC = 256
OUT = 10

NC = 2
NS = 16
NW = NC * NS
EPT = E // NW
CHUNK = 80
NCHUNK = EPT // CHUNK
RPT = 624
RPT0 = 16
NLANE = D // 16


def _make_spmm(interpret=False):
  mesh = plsc.VectorSubcoreMesh(core_axis_name="c", subcore_axis_name="s")

  @functools.partial(
      pl.kernel,
      out_type=jax.ShapeDtypeStruct((NC, N, D), jnp.float32),
      mesh=mesh,
      scratch_types=[
          pltpu.VMEM((EPT,), jnp.int32),
          pltpu.VMEM((CHUNK,), jnp.float32),
          pltpu.VMEM((CHUNK,), jnp.float32),
          pltpu.VMEM((CHUNK,), jnp.float32),
          pltpu.VMEM((CHUNK,), jnp.int32),
          pltpu.VMEM((CHUNK,), jnp.int32),
          pltpu.VMEM((CHUNK,), jnp.int32),
          pltpu.VMEM((CHUNK,), jnp.int32),
          pltpu.VMEM((CHUNK,), jnp.int32),
          pltpu.VMEM((CHUNK,), jnp.int32),
          pltpu.VMEM((CHUNK, D), jnp.float32),
          pltpu.VMEM((CHUNK, D), jnp.float32),
          pltpu.VMEM((CHUNK, D), jnp.float32),
          pltpu.VMEM_SHARED((N, D), jnp.float32),
          pltpu.SemaphoreType.DMA,
          pltpu.SemaphoreType.DMA,
          pltpu.SemaphoreType.DMA,
          pltpu.SemaphoreType.DMA,
          pltpu.SemaphoreType.DMA,
          pltpu.SemaphoreType.DMA,
          pltpu.SemaphoreType.DMA,
      ],
      interpret=interpret,
  )
  def spmm(x_hbm, pk_hbm, w_hbm, out_hbm,
           pk_v, w0, w1, w2, src0, src1, src2, dst0, dst1, dst2,
           rows0, rows1, rows2, acc,
           semi, semg0, semg1, semg2, sems0, sems1, sems2):
    c = lax.axis_index("c")
    s = lax.axis_index("s")
    wid = c * NS + s
    ebase = wid * EPT

    dpk = pltpu.async_copy(pk_hbm.at[pl.ds(ebase, EPT)], pk_v, semi)

    zero16 = jnp.zeros((16,), jnp.float32)
    wv_ = (w0, w1, w2)
    srcc = (src0, src1, src2)
    dstc = (dst0, dst1, dst2)
    rows = (rows0, rows1, rows2)
    semg = (semg0, semg1, semg2)
    sems = (sems0, sems1, sems2)

    @pl.loop(0, CHUNK)
    def _zero_fill(r):
      for j in range(NLANE):
        rows0[r, pl.ds(j * 16, 16)] = zero16
        rows1[r, pl.ds(j * 16, 16)] = zero16
        rows2[r, pl.ds(j * 16, 16)] = zero16

    row0 = RPT0 + s * RPT
    zcopies = []
    for k in range(RPT // CHUNK):
      zcopies.append(pltpu.async_copy(
          rows0, acc.at[pl.ds(row0 + k * CHUNK, CHUNK)], semg0))
    rem = RPT - (RPT // CHUNK) * CHUNK
    zcopies.append(pltpu.async_copy(
        rows0.at[pl.ds(0, rem)], acc.at[pl.ds(row0 + RPT - rem, rem)],
        semg0))

    @pl.when(s == 0)
    def _zero_head():
      pltpu.sync_copy(rows0.at[pl.ds(0, RPT0)], acc.at[pl.ds(0, RPT0)])

    for d in zcopies:
      d.wait()
    dpk.wait()
    plsc.subcore_barrier()

    def unpack(k, r):
      for g in range(CHUNK // 16):
        v = pk_v[pl.ds(k * CHUNK + g * 16, 16)]
        sl = pl.ds(g * 16, 16)
        srcc[r][sl] = jnp.bitwise_and(v, 0xFFFF)
        dstc[r][sl] = lax.shift_right_logical(v, 16)

    def issue_gather(k, r):
      pltpu.async_copy(x_hbm.at[srcc[r]], rows[r], semg[r])
      pltpu.async_copy(w_hbm.at[pl.ds(ebase + k * CHUNK, CHUNK)],
                       wv_[r], semg[r])

    def wait_gather(r):
      pltpu.make_async_copy(x_hbm.at[pl.ds(0, CHUNK)], rows[r],
                            semg[r]).wait()
      pltpu.make_async_copy(w_hbm.at[pl.ds(0, CHUNK)], wv_[r],
                            semg[r]).wait()

    def issue_scatter(r):
      pltpu.async_copy(rows[r], acc.at[dstc[r]], sems[r], add=True)

    def wait_scatter(r):
      pltpu.make_async_copy(rows[r], acc.at[dstc[r]], sems[r]).wait()

    def scale(r):
      rbuf = rows[r]
      wref = wv_[r]

      @pl.loop(0, CHUNK // 16)
      def _scale(g):
        wgrp = wref[pl.ds(g * 16, 16)]
        for e in range(16):
          wv = wgrp.at[jnp.full((16,), e, jnp.int32)].get(
              mode="promise_in_bounds")
          i = g * 16 + e
          for j in range(NLANE):
            sl = pl.ds(j * 16, 16)
            rbuf[i, sl] = rbuf[i, sl] * wv

    def step(k, r, rn):
      wait_scatter(rn)
      unpack(k + 1, rn)
      issue_gather(k + 1, rn)
      wait_gather(r)
      scale(r)
      issue_scatter(r)

    unpack(0, 1)
    issue_scatter(1)
    unpack(0, 2)
    issue_scatter(2)
    unpack(0, 0)
    issue_gather(0, 0)

    @pl.loop(0, (NCHUNK - 2) // 3)
    def _tri(i):
      k0 = i * 3
      step(k0, 0, 1)
      step(k0 + 1, 1, 2)
      step(k0 + 2, 2, 0)

    step(NCHUNK - 2, 0, 1)
    wait_scatter(2)
    wait_gather(1)
    scale(1)
    issue_scatter(1)
    wait_scatter(0)
    wait_scatter(1)

    plsc.subcore_barrier()
    pltpu.sync_copy(acc.at[pl.ds(row0, RPT)],
                    out_hbm.at[c, pl.ds(row0, RPT)])

    @pl.when(s == 0)
    def _write_head():
      pltpu.sync_copy(acc.at[pl.ds(0, RPT0)], out_hbm.at[c, pl.ds(0, RPT0)])

  return spmm


def _dense1_body(aggp, x, wrel, brel, wroot, g, b, out):
  agg = aggp[0] + aggp[1]
  z = lax.dot_general(agg, wrel[...], (((1,), (1,)), ((), ())),
                      precision=lax.Precision.HIGHEST)
  z = z + brel[...]
  z = z + lax.dot_general(x[...], wroot[...], (((1,), (1,)), ((), ())),
                          precision=lax.Precision.HIGHEST)
  z = jnp.maximum(z, 0.0)
  mu = jnp.mean(z, axis=0, keepdims=True)
  var = jnp.mean((z - mu) ** 2, axis=0, keepdims=True)
  out[...] = g[...] * (z - mu) / jnp.sqrt(var + 1e-5) + b[...]


def _dense2_body(aggp, h1, wrel, brel, wroot, g2, b2, mem2d,
                 fc1w, fc1b, g3, b3, fc2w, fc2b, out):
  agg = aggp[0] + aggp[1]
  z = lax.dot_general(agg, wrel[...], (((1,), (1,)), ((), ())),
                      precision=lax.Precision.HIGHEST)
  z = z + brel[...]
  z = z + lax.dot_general(h1[...], wroot[...], (((1,), (1,)), ((), ())),
                          precision=lax.Precision.HIGHEST)
  z = jnp.maximum(z, 0.0)
  mu = jnp.mean(z, axis=0, keepdims=True)
  var = jnp.mean((z - mu) ** 2, axis=0, keepdims=True)
  h2 = g2[...] * (z - mu) / jnp.sqrt(var + 1e-5) + b2[...]

  gids = lax.broadcasted_iota(jnp.int32, (N, G), 1)
  oh = (mem2d[...] == gids).astype(jnp.float32)
  ssum = lax.dot_general(oh, h2, (((0,), (0,)), ((), ())),
                         precision=lax.Precision.HIGHEST)
  cnt = lax.dot_general(oh, jnp.ones((N, 1), jnp.float32),
                        (((0,), (0,)), ((), ())),
                        precision=lax.Precision.HIGHEST)
  pooled = ssum / jnp.maximum(cnt, 1.0)

  a = lax.dot_general(pooled, fc1w[...], (((1,), (1,)), ((), ())),
                      precision=lax.Precision.HIGHEST)
  a = jnp.maximum(a + fc1b[...], 0.0)
  mu3 = jnp.mean(a, axis=0, keepdims=True)
  var3 = jnp.mean((a - mu3) ** 2, axis=0, keepdims=True)
  a = g3[...] * (a - mu3) / jnp.sqrt(var3 + 1e-5) + b3[...]

  logits = lax.dot_general(a, fc2w[...], (((1,), (1,)), ((), ())),
                           precision=lax.Precision.HIGHEST)
  out[...] = logits + fc2b[...]


_TC_PARAMS = pltpu.CompilerParams(vmem_limit_bytes=100 * 1024 * 1024)


def _make_dense1(interpret=False):
  return pl.pallas_call(
      _dense1_body,
      out_shape=jax.ShapeDtypeStruct((N, D), jnp.float32),
      compiler_params=_TC_PARAMS,
      interpret=interpret,
  )


def _make_dense2(interpret=False):
  return pl.pallas_call(
      _dense2_body,
      out_shape=jax.ShapeDtypeStruct((G, OUT), jnp.float32),
      compiler_params=_TC_PARAMS,
      interpret=interpret,
  )


def kernel(x, membership, edges, weights, W_rel1, b_rel1, W_root1,
           W_rel2, b_rel2, W_root2, bn1_g, bn1_b, bn2_g, bn2_b,
           fc1_W, fc1_b, bn3_g, bn3_b, fc2_W, fc2_b):
  packed = jnp.bitwise_or(edges[0], jnp.left_shift(edges[1], 16))
  mem2d = membership.reshape(N, 1)

  spmm = _make_spmm()
  dense1 = _make_dense1()
  dense2 = _make_dense2()

  aggp1 = spmm(x, packed, weights)
  h1 = dense1(aggp1, x, W_rel1, b_rel1.reshape(1, D), W_root1,
              bn1_g.reshape(1, D), bn1_b.reshape(1, D))
  aggp2 = spmm(h1, packed, weights)
  logits = dense2(aggp2, h1, W_rel2, b_rel2.reshape(1, D), W_root2,
                  bn2_g.reshape(1, D), bn2_b.reshape(1, D), mem2d,
                  fc1_W, fc1_b.reshape(1, FC), bn3_g.reshape(1, FC),
                  bn3_b.reshape(1, FC), fc2_W, fc2_b.reshape(1, OUT))
  return logits
